# baseline (speedup 1.0000x reference)
import functools

import jax
import jax.numpy as jnp
from jax import lax
from jax.experimental import pallas as pl
from jax.experimental.pallas import tpu as pltpu
from jax.experimental.pallas import tpu_sc as plsc

B = 1024
L = 20
C = 1000
SLAB = C * L
NC, NS = 2, 16
NW = NC * NS
BPW = B // NW
NBUF = 2


def _sc_body(t_hbm, z_hbm, out_hbm, idx_v, slab0, slab1, sem0, sem1):
    wid = lax.axis_index("s") * NC + lax.axis_index("c")
    ibase = wid * (BPW * L)
    obase = wid * (BPW * SLAB)

    pltpu.sync_copy(t_hbm.at[pl.ds(ibase, BPW * L)], idx_v.at[pl.ds(0, BPW * L)])
    pltpu.sync_copy(z_hbm, slab0)
    pltpu.sync_copy(z_hbm, slab1)

    lane = lax.iota(jnp.int32, 16)
    tail_mask = lane < (L - 16)
    one_v = jnp.ones((16,), jnp.float32)
    zero_v = jnp.zeros((16,), jnp.float32)

    slabs = (slab0, slab1)
    sems = (sem0, sem1)

    def offsets(b):
        head = idx_v[pl.ds(b * L, 16)] * L + lane
        tail = idx_v[pl.ds(b * L + 16, 16)] * L + (16 + lane)
        return head, tail

    def scatter(slab, head, tail, val):
        plsc.store_scatter(slab, [head], val)
        plsc.store_scatter(slab, [tail], val, mask=tail_mask)

    pending = [None, None]
    prev_offs = [None, None]
    for b in range(BPW):
        s = b % NBUF
        if pending[s] is not None:
            pending[s].wait()
            scatter(slabs[s], *prev_offs[s], zero_v)
        head, tail = offsets(b)
        scatter(slabs[s], head, tail, one_v)
        pending[s] = pltpu.async_copy(
            slabs[s], out_hbm.at[pl.ds(obase + b * SLAB, SLAB)], sems[s])
        prev_offs[s] = (head, tail)
    for s in range(NBUF):
        pending[s].wait()


@functools.partial(jax.jit, static_argnames=())
def _one_hot_sc(t_flat, zeros_src):
    mesh = plsc.VectorSubcoreMesh(core_axis_name="c", subcore_axis_name="s")
    run = pl.kernel(
        _sc_body,
        out_type=jax.ShapeDtypeStruct((B * C * L,), jnp.float32),
        mesh=mesh,
        scratch_types=[
            pltpu.VMEM((BPW * L + 16,), jnp.int32),
            pltpu.VMEM((SLAB,), jnp.float32),
            pltpu.VMEM((SLAB,), jnp.float32),
            pltpu.SemaphoreType.DMA,
            pltpu.SemaphoreType.DMA,
        ],
        compiler_params=pltpu.CompilerParams(needs_layout_passes=False),
        name="one_hot_sc",
    )
    return run(t_flat, zeros_src)


def kernel(t, ones):
    del ones
    t_flat = t.reshape(-1).astype(jnp.int32)
    zeros_src = jnp.zeros((SLAB,), jnp.float32)
    out_flat = _one_hot_sc(t_flat, zeros_src)
    return out_flat.reshape(B, C, L)


# speedup vs baseline: 18.1792x; 18.1792x over previous
import functools

import jax
import jax.numpy as jnp
from jax import lax
from jax.experimental import pallas as pl
from jax.experimental.pallas import tpu as pltpu
from jax.experimental.pallas import tpu_sc as plsc

B = 1024
L = 20
C = 1000
FLAT = B * C * L
NC, NS = 2, 16
BPS = B // NS
LPC = L // NC
ZCH = 80000
NZD = FLAT // (NC * NS) // ZCH


def _sc_body(t_hbm, z_hbm, one_hbm, out_hbm, t_v, offs_v, ones_v, zslab,
             sem_z, sem_s):
    core = lax.axis_index("c")
    sub = lax.axis_index("s")
    wid = core * NS + sub

    pltpu.sync_copy(z_hbm, zslab)
    zbase = wid * (NZD * ZCH)
    zcopies = [
        pltpu.async_copy(zslab, out_hbm.at[pl.ds(zbase + k * ZCH, ZCH)], sem_z)
        for k in range(NZD)
    ]

    pltpu.sync_copy(t_hbm.at[pl.ds(sub * (BPS * L), BPS * L)], t_v)
    pltpu.sync_copy(one_hbm, ones_v)

    lane = lax.iota(jnp.int32, 16)
    for lr in range(LPC):
        l_abs = core * LPC + lr
        for k in range(4):
            gidx = (k * 16 + lane) * L + l_abs
            vals = plsc.load_gather(t_v, [gidx])
            off = (
                l_abs * (C * B)
                + (sub >> 1) * 1024
                + (sub & 1) * 64
                + k * 16
                + (vals >> 3) * 8192
                + (vals & 7) * 128
                + lane
            )
            j = lr * 4 + k
            offs_v[j // 8, pl.ds((j % 8) * 16, 16)] = off

    for cp in zcopies:
        cp.wait()
    plsc.subcore_barrier()

    scopies = [
        pltpu.async_copy(ones_v, out_hbm.at[offs_v.at[j]], sem_s)
        for j in range(5)
    ]
    for cp in scopies:
        cp.wait()


@jax.jit
def _one_hot_sc(t_flat, zeros_src, ones_src):
    mesh = plsc.VectorSubcoreMesh(core_axis_name="c", subcore_axis_name="s")
    run = pl.kernel(
        _sc_body,
        out_type=jax.ShapeDtypeStruct((FLAT,), jnp.float32),
        mesh=mesh,
        scratch_types=[
            pltpu.VMEM((BPS * L,), jnp.int32),
            pltpu.VMEM((5, 128), jnp.int32),
            pltpu.VMEM((128,), jnp.float32),
            pltpu.VMEM((ZCH,), jnp.float32),
            pltpu.SemaphoreType.DMA,
            pltpu.SemaphoreType.DMA,
        ],
        compiler_params=pltpu.CompilerParams(needs_layout_passes=False),
        name="one_hot_sc",
    )
    flat = run(t_flat, zeros_src, ones_src)
    return (
        flat.reshape(L, C // 8, B // 128, 8, 128)
        .transpose(2, 4, 1, 3, 0)
        .reshape(B, C, L)
    )


def kernel(t, ones):
    del ones
    t_flat = t.reshape(-1).astype(jnp.int32)
    zeros_src = jnp.zeros((ZCH,), jnp.float32)
    ones_src = jnp.ones((128,), jnp.float32)
    return _one_hot_sc(t_flat, zeros_src, ones_src)


# speedup vs baseline: 19.1199x; 1.0517x over previous
import functools

import jax
import jax.numpy as jnp
from jax import lax
from jax.experimental import pallas as pl
from jax.experimental.pallas import tpu as pltpu
from jax.experimental.pallas import tpu_sc as plsc

B = 1024
L = 20
C = 1000
FLAT = B * C * L
NC, NS = 2, 16
BPS = B // NS
LPC = L // NC
ZCH = 40000
NZD = FLAT // (NC * NS) // ZCH


def _sc_body(t_hbm, z_hbm, one_hbm, out_hbm, t_v, offs_v, ones_v, zslab,
             sem_z, sem_s):
    core = lax.axis_index("c")
    sub = lax.axis_index("s")
    wid = core * NS + sub

    pltpu.sync_copy(z_hbm, zslab)
    zbase = wid * (NZD * ZCH)
    zcopies = [
        pltpu.async_copy(zslab, out_hbm.at[pl.ds(zbase + k * ZCH, ZCH)], sem_z)
        for k in range(NZD)
    ]

    pltpu.sync_copy(t_hbm.at[pl.ds(sub * (BPS * L), BPS * L)], t_v)
    pltpu.sync_copy(one_hbm, ones_v)

    lane = lax.iota(jnp.int32, 16)
    for lr in range(LPC):
        l_abs = core * LPC + lr
        for k in range(4):
            gidx = (k * 16 + lane) * L + l_abs
            vals = plsc.load_gather(t_v, [gidx])
            off = (
                l_abs * (C * B)
                + (sub >> 1) * 1024
                + (sub & 1) * 64
                + k * 16
                + (vals >> 3) * 8192
                + (vals & 7) * 128
                + lane
            )
            j = lr * 4 + k
            offs_v[j // 8, pl.ds((j % 8) * 16, 16)] = off

    for cp in zcopies:
        cp.wait()
    plsc.subcore_barrier()

    scopies = [
        pltpu.async_copy(ones_v, out_hbm.at[offs_v.at[j]], sem_s)
        for j in range(5)
    ]
    for cp in scopies:
        cp.wait()


@jax.jit
def _one_hot_sc(t_flat, zeros_src, ones_src):
    mesh = plsc.VectorSubcoreMesh(core_axis_name="c", subcore_axis_name="s")
    run = pl.kernel(
        _sc_body,
        out_type=jax.ShapeDtypeStruct((FLAT,), jnp.float32),
        mesh=mesh,
        scratch_types=[
            pltpu.VMEM((BPS * L,), jnp.int32),
            pltpu.VMEM((5, 128), jnp.int32),
            pltpu.VMEM((128,), jnp.float32),
            pltpu.VMEM((ZCH,), jnp.float32),
            pltpu.SemaphoreType.DMA,
            pltpu.SemaphoreType.DMA,
        ],
        compiler_params=pltpu.CompilerParams(needs_layout_passes=False),
        name="one_hot_sc",
    )
    flat = run(t_flat, zeros_src, ones_src)
    return (
        flat.reshape(L, C // 8, B // 128, 8, 128)
        .transpose(2, 4, 1, 3, 0)
        .reshape(B, C, L)
    )


def kernel(t, ones):
    del ones
    t_flat = t.reshape(-1).astype(jnp.int32)
    zeros_src = jnp.zeros((ZCH,), jnp.float32)
    ones_src = jnp.ones((128,), jnp.float32)
    return _one_hot_sc(t_flat, zeros_src, ones_src)


# speedup vs baseline: 19.2750x; 1.0081x over previous
import functools

import jax
import jax.numpy as jnp
from jax import lax
from jax.experimental import pallas as pl
from jax.experimental.pallas import tpu as pltpu
from jax.experimental.pallas import tpu_sc as plsc

B = 1024
L = 20
C = 1000
FLAT = B * C * L
NC, NS = 2, 16
BPS = B // NS
LPC = L // NC
ZCH = 20000
NZD = FLAT // (NC * NS) // ZCH


def _sc_body(t_hbm, z_hbm, one_hbm, out_hbm, t_v, offs_v, ones_v, zslab,
             sem_z, sem_s):
    core = lax.axis_index("c")
    sub = lax.axis_index("s")
    wid = core * NS + sub

    pltpu.sync_copy(z_hbm, zslab)
    zbase = wid * (NZD * ZCH)
    zcopies = [
        pltpu.async_copy(zslab, out_hbm.at[pl.ds(zbase + k * ZCH, ZCH)], sem_z)
        for k in range(NZD)
    ]

    pltpu.sync_copy(t_hbm.at[pl.ds(sub * (BPS * L), BPS * L)], t_v)
    pltpu.sync_copy(one_hbm, ones_v)

    lane = lax.iota(jnp.int32, 16)
    for lr in range(LPC):
        l_abs = core * LPC + lr
        for k in range(4):
            gidx = (k * 16 + lane) * L + l_abs
            vals = plsc.load_gather(t_v, [gidx])
            off = (
                l_abs * (C * B)
                + (sub >> 1) * 1024
                + (sub & 1) * 64
                + k * 16
                + (vals >> 3) * 8192
                + (vals & 7) * 128
                + lane
            )
            j = lr * 4 + k
            offs_v[j // 8, pl.ds((j % 8) * 16, 16)] = off

    for cp in zcopies:
        cp.wait()
    plsc.subcore_barrier()

    scopies = [
        pltpu.async_copy(ones_v, out_hbm.at[offs_v.at[j]], sem_s)
        for j in range(5)
    ]
    for cp in scopies:
        cp.wait()


@jax.jit
def _one_hot_sc(t_flat, zeros_src, ones_src):
    mesh = plsc.VectorSubcoreMesh(core_axis_name="c", subcore_axis_name="s")
    run = pl.kernel(
        _sc_body,
        out_type=jax.ShapeDtypeStruct((FLAT,), jnp.float32),
        mesh=mesh,
        scratch_types=[
            pltpu.VMEM((BPS * L,), jnp.int32),
            pltpu.VMEM((5, 128), jnp.int32),
            pltpu.VMEM((128,), jnp.float32),
            pltpu.VMEM((ZCH,), jnp.float32),
            pltpu.SemaphoreType.DMA,
            pltpu.SemaphoreType.DMA,
        ],
        compiler_params=pltpu.CompilerParams(needs_layout_passes=False),
        name="one_hot_sc",
    )
    flat = run(t_flat, zeros_src, ones_src)
    return (
        flat.reshape(L, C // 8, B // 128, 8, 128)
        .transpose(2, 4, 1, 3, 0)
        .reshape(B, C, L)
    )


def kernel(t, ones):
    del ones
    t_flat = t.reshape(-1).astype(jnp.int32)
    zeros_src = jnp.zeros((ZCH,), jnp.float32)
    ones_src = jnp.ones((128,), jnp.float32)
    return _one_hot_sc(t_flat, zeros_src, ones_src)
